# baseline (speedup 1.0000x reference)
import functools

import jax
import jax.numpy as jnp
from jax import lax
from jax.experimental import pallas as pl
from jax.experimental.pallas import tpu as pltpu
from jax.experimental.pallas import tpu_sc as plsc

_TABLE_SIZE = 1000000
_SUB_DIM = 32
_EMBED_DIM = 128
_HASH_SEED = 42

_BATCH = 16384
_FIELDS = 26
_N = _BATCH * _FIELDS

_NC = 2
_NS = 16
_NW = _NC * _NS
_PER_W = _N // _NW
_G = 4 * _FIELDS
_NGRP = _PER_W // _G
_BPW = _BATCH // _NW
_SUB = 2
_BSUB = _BPW // _SUB

_TR = 2048
_NBLK = 489
_XROWS = _NBLK * _TR
_XV_ROWS = 4 * _XROWS


def _repack_body(q0, q1, q2, q3, out_ref):
    out_ref[...] = jnp.concatenate([q0[...], q1[...], q2[...], q3[...]], axis=0).T


def _repack4(tts):
    spec = pl.BlockSpec((_SUB_DIM, _TR), lambda i: (0, i))
    return pl.pallas_call(
        _repack_body,
        grid=(_NBLK,),
        in_specs=[spec] * 4,
        out_specs=pl.BlockSpec((_TR, _EMBED_DIM), lambda i: (i, 0)),
        out_shape=jax.ShapeDtypeStruct((_XROWS, _EMBED_DIM), jnp.float32),
    )(*tts)


def _hash_body(ids_ref, o0, o1, o2, o3):
    x = ids_ref[...].astype(jnp.uint32)
    outs = (o0, o1, o2, o3)
    for t in range(4):
        h = x ^ jnp.uint32(_HASH_SEED + t)
        h = h * jnp.uint32(2654435761)
        h = h ^ (h >> jnp.uint32(16))
        h = h * jnp.uint32(2246822519)
        h = h ^ (h >> jnp.uint32(13))
        h = (h % jnp.uint32(_TABLE_SIZE)).astype(jnp.int32)
        g = (h << 2) | t
        outs[t][...] = g.reshape((_N,))


def _hash_ids_tc(ids_t):
    shp = jax.ShapeDtypeStruct((_N,), jnp.int32)
    return pl.pallas_call(
        _hash_body,
        out_shape=[shp, shp, shp, shp],
    )(ids_t)


def _sc_body(i0, i1, i2, i3, xv, out_hbm,
             slabs, idx_v, bufs, ssem, gsem, wsem):
    wid = lax.axis_index("s") * _NC + lax.axis_index("c")
    base = wid * _PER_W
    b0w = wid * _BPW
    idxin = (i0, i1, i2, i3)
    tables = (xv, xv, xv, xv)

    lane = lax.iota(jnp.int32, 16)
    for sub in range(_SUB):
        b0 = b0w + sub * _BSUB
        descs = [
            pltpu.async_copy(
                idxin[t].at[:, pl.ds(b0, _BSUB)], slabs.at[t], ssem
            )
            for t in range(4)
        ]
        for d in descs:
            d.wait()

        def trans_body(f, carry, sub=sub):
            for t in range(4):
                for bb in range(_BSUB // 16):
                    v = slabs[t, f, pl.ds(bb * 16, 16)]
                    bloc = lane + sub * _BSUB + bb * 16
                    plsc.store_scatter(
                        idx_v,
                        [jnp.full((16,), t, jnp.int32),
                         bloc >> jnp.int32(2),
                         (bloc & jnp.int32(3)) * _FIELDS + f],
                        v,
                    )
            return carry

        lax.fori_loop(0, _FIELDS, trans_body, 0)

    wait_dst = out_hbm.at[0, :, pl.ds(0, _SUB_DIM)]

    def wait_buf():
        for _ in range(4):
            pltpu.make_async_copy(
                bufs.at[0, pl.ds(0, _FIELDS), :], wait_dst, wsem
            ).wait()

    def do_round(t, table, r, half, skip_wait):
        if not skip_wait:
            for b in range(4):
                wait_buf()
        descs = []
        for b in range(4):
            g = r * 4 + b
            descs.append(
                pltpu.async_copy(table.at[idx_v.at[t, g]], bufs.at[half * 4 + b], gsem)
            )
        for d in descs:
            d.wait()
        for b in range(4):
            g = r * 4 + b
            for k in range(4):
                pltpu.async_copy(
                    bufs.at[half * 4 + b, pl.ds(k * _FIELDS, _FIELDS), :],
                    out_hbm.at[b0w + g * 4 + k, :, pl.ds(t * _SUB_DIM, _SUB_DIM)],
                    wsem,
                )

    nr = _NGRP // 4
    for t in range(4):
        table = tables[t]
        if t == 0:
            do_round(t, table, 0, 0, True)
            do_round(t, table, 1, 1, True)

            def body0(rr, carry):
                do_round(t, table, 2 + 2 * rr, 0, False)
                do_round(t, table, 3 + 2 * rr, 1, False)
                return carry

            lax.fori_loop(0, (nr - 2) // 2, body0, 0)
        else:

            def body(rr, carry, t=t, table=table):
                do_round(t, table, 2 * rr, 0, False)
                do_round(t, table, 2 * rr + 1, 1, False)
                return carry

            lax.fori_loop(0, nr // 2, body, 0)

    for b in range(8):
        wait_buf()


def _sc_gather(idx2, xv):
    mesh = plsc.VectorSubcoreMesh(core_axis_name="c", subcore_axis_name="s")
    k = pl.kernel(
        _sc_body,
        out_type=jax.ShapeDtypeStruct((_BATCH, _FIELDS, _EMBED_DIM), jnp.float32),
        mesh=mesh,
        compiler_params=pltpu.CompilerParams(
            use_tc_tiling_on_sc=False, needs_layout_passes=False
        ),
        scratch_types=[
            pltpu.VMEM((4, _FIELDS, _BSUB), jnp.int32),
            pltpu.VMEM((4, _NGRP, _G), jnp.int32),
            pltpu.VMEM((8, _G, _SUB_DIM), jnp.float32),
            pltpu.SemaphoreType.DMA,
            pltpu.SemaphoreType.DMA,
            pltpu.SemaphoreType.DMA,
        ],
    )
    return k(*idx2, xv)


@jax.jit
def _bloom_embed(input_ids, t0, t1, t2, t3):
    idx1 = _hash_ids_tc(input_ids.T)
    idx2 = [v.reshape(_FIELDS, _BATCH) for v in idx1]
    xv = _repack4([t.T for t in (t0, t1, t2, t3)]).reshape(_XV_ROWS, _SUB_DIM)
    return _sc_gather(idx2, xv)


def kernel(input_ids, table0, table1, table2, table3):
    return _bloom_embed(
        input_ids.astype(jnp.int32), table0, table1, table2, table3
    )


# speedup vs baseline: 1.1644x; 1.1644x over previous
import functools

import jax
import jax.numpy as jnp
from jax import lax
from jax.experimental import pallas as pl
from jax.experimental.pallas import tpu as pltpu
from jax.experimental.pallas import tpu_sc as plsc

_TABLE_SIZE = 1000000
_SUB_DIM = 32
_EMBED_DIM = 128
_HASH_SEED = 42

_BATCH = 16384
_FIELDS = 26
_N = _BATCH * _FIELDS

_NC = 2
_NS = 16
_NW = _NC * _NS
_PER_W = _N // _NW
_G = 128
_NGRP = _PER_W // _G
_BPW = _BATCH // _NW
_SUB = 2
_BSUB = _BPW // _SUB

_TR = 4096
_NBLK = 245
_XROWS = _NBLK * _TR
_XV_ROWS = 4 * _XROWS


def _repack_body(q0, q1, q2, q3, out_ref):
    out_ref[...] = jnp.concatenate([q0[...], q1[...], q2[...], q3[...]], axis=0).T


def _repack4(tts):
    spec = pl.BlockSpec((_SUB_DIM, _TR), lambda i: (0, i))
    return pl.pallas_call(
        _repack_body,
        grid=(_NBLK,),
        in_specs=[spec] * 4,
        out_specs=pl.BlockSpec((_TR, _EMBED_DIM), lambda i: (i, 0)),
        out_shape=jax.ShapeDtypeStruct((_XROWS, _EMBED_DIM), jnp.float32),
    )(*tts)


def _hash_body(ids_ref, o0, o1, o2, o3):
    x = ids_ref[...].astype(jnp.uint32)
    outs = (o0, o1, o2, o3)
    for t in range(4):
        h = x ^ jnp.uint32(_HASH_SEED + t)
        h = h * jnp.uint32(2654435761)
        h = h ^ (h >> jnp.uint32(16))
        h = h * jnp.uint32(2246822519)
        h = h ^ (h >> jnp.uint32(13))
        h = (h % jnp.uint32(_TABLE_SIZE)).astype(jnp.int32)
        g = (h << 2) | t
        outs[t][...] = g.reshape((_N,))


def _hash_ids_tc(ids_t):
    shp = jax.ShapeDtypeStruct((_N,), jnp.int32)
    return pl.pallas_call(
        _hash_body,
        out_shape=[shp, shp, shp, shp],
    )(ids_t)


def _sc_body(i0, i1, i2, i3, xv, out_hbm,
             slabs, idx_v, bufs, ssem, gsem, wsem):
    wid = lax.axis_index("s") * _NC + lax.axis_index("c")
    base = wid * _PER_W
    b0w = wid * _BPW
    idxin = (i0, i1, i2, i3)
    tables = (xv, xv, xv, xv)

    lane = lax.iota(jnp.int32, 16)
    for sub in range(_SUB):
        b0 = b0w + sub * _BSUB
        descs = [
            pltpu.async_copy(
                idxin[t].at[:, pl.ds(b0, _BSUB)], slabs.at[t], ssem
            )
            for t in range(4)
        ]
        for d in descs:
            d.wait()

        def trans_body(f, carry, sub=sub):
            for t in range(4):
                for bb in range(_BSUB // 16):
                    v = slabs[t, f, pl.ds(bb * 16, 16)]
                    nloc = (lane + sub * _BSUB + bb * 16) * _FIELDS + f
                    plsc.store_scatter(
                        idx_v,
                        [jnp.full((16,), t, jnp.int32),
                         nloc >> jnp.int32(7),
                         nloc & jnp.int32(127)],
                        v,
                    )
            return carry

        lax.fori_loop(0, _FIELDS, trans_body, 0)

    wait_dst = out_hbm.at[pl.ds(0, _G), pl.ds(0, _SUB_DIM)]

    def do_round(t, table, r, half, skip_wait):
        if not skip_wait:
            for b in range(4):
                pltpu.make_async_copy(bufs.at[half * 4 + b], wait_dst, wsem).wait()
        descs = []
        for b in range(4):
            g = r * 4 + b
            descs.append(
                pltpu.async_copy(table.at[idx_v.at[t, g]], bufs.at[half * 4 + b], gsem)
            )
        for d in descs:
            d.wait()
        for b in range(4):
            g = r * 4 + b
            row0 = base + g * _G
            pltpu.async_copy(
                bufs.at[half * 4 + b],
                out_hbm.at[pl.ds(row0, _G), pl.ds(t * _SUB_DIM, _SUB_DIM)],
                wsem,
            )

    nr = _NGRP // 4
    for t in range(4):
        table = tables[t]
        if t == 0:
            do_round(t, table, 0, 0, True)
            do_round(t, table, 1, 1, True)

            def body0(rr, carry):
                do_round(t, table, 2 + 2 * rr, 0, False)
                do_round(t, table, 3 + 2 * rr, 1, False)
                return carry

            lax.fori_loop(0, (nr - 2) // 2, body0, 0)
        else:

            def body(rr, carry, t=t, table=table):
                do_round(t, table, 2 * rr, 0, False)
                do_round(t, table, 2 * rr + 1, 1, False)
                return carry

            lax.fori_loop(0, nr // 2, body, 0)

    for b in range(8):
        pltpu.make_async_copy(bufs.at[b], wait_dst, wsem).wait()


def _sc_gather(idx2, xv):
    mesh = plsc.VectorSubcoreMesh(core_axis_name="c", subcore_axis_name="s")
    k = pl.kernel(
        _sc_body,
        out_type=jax.ShapeDtypeStruct((_N, _EMBED_DIM), jnp.float32),
        mesh=mesh,
        compiler_params=pltpu.CompilerParams(
            use_tc_tiling_on_sc=False, needs_layout_passes=False
        ),
        scratch_types=[
            pltpu.VMEM((4, _FIELDS, _BSUB), jnp.int32),
            pltpu.VMEM((4, _NGRP, _G), jnp.int32),
            pltpu.VMEM((8, _G, _SUB_DIM), jnp.float32),
            pltpu.SemaphoreType.DMA,
            pltpu.SemaphoreType.DMA,
            pltpu.SemaphoreType.DMA,
        ],
    )
    return k(*idx2, xv)


@jax.jit
def _bloom_embed(input_ids, t0, t1, t2, t3):
    idx1 = _hash_ids_tc(input_ids.T)
    idx2 = [v.reshape(_FIELDS, _BATCH) for v in idx1]
    xv = _repack4([t.T for t in (t0, t1, t2, t3)]).reshape(_XV_ROWS, _SUB_DIM)
    out = _sc_gather(idx2, xv)
    return out.reshape(_BATCH, _FIELDS, _EMBED_DIM)


def kernel(input_ids, table0, table1, table2, table3):
    return _bloom_embed(
        input_ids.astype(jnp.int32), table0, table1, table2, table3
    )


# speedup vs baseline: 1.2280x; 1.0546x over previous
import functools

import jax
import jax.numpy as jnp
from jax import lax
from jax.experimental import pallas as pl
from jax.experimental.pallas import tpu as pltpu
from jax.experimental.pallas import tpu_sc as plsc

_TABLE_SIZE = 1000000
_SUB_DIM = 32
_EMBED_DIM = 128
_HASH_SEED = 42

_BATCH = 16384
_FIELDS = 26
_N = _BATCH * _FIELDS

_NC = 2
_NS = 16
_NW = _NC * _NS
_PER_W = _N // _NW
_G = 128
_NGRP = _PER_W // _G
_BPW = _BATCH // _NW
_SUB = 2
_BSUB = _BPW // _SUB

_TR = 8192
_NBLK = 123
_XROWS = _NBLK * _TR
_XV_ROWS = 4 * _XROWS


def _repack_body(q0, q1, q2, q3, out_ref):
    out_ref[...] = jnp.concatenate([q0[...], q1[...], q2[...], q3[...]], axis=0).T


def _repack4(tts):
    spec = pl.BlockSpec((_SUB_DIM, _TR), lambda i: (0, i))
    return pl.pallas_call(
        _repack_body,
        grid=(_NBLK,),
        in_specs=[spec] * 4,
        out_specs=pl.BlockSpec((_TR, _EMBED_DIM), lambda i: (i, 0)),
        out_shape=jax.ShapeDtypeStruct((_XROWS, _EMBED_DIM), jnp.float32),
    )(*tts)


def _hash_body(ids_ref, o0, o1, o2, o3):
    x = ids_ref[...].astype(jnp.uint32)
    outs = (o0, o1, o2, o3)
    for t in range(4):
        h = x ^ jnp.uint32(_HASH_SEED + t)
        h = h * jnp.uint32(2654435761)
        h = h ^ (h >> jnp.uint32(16))
        h = h * jnp.uint32(2246822519)
        h = h ^ (h >> jnp.uint32(13))
        h = (h % jnp.uint32(_TABLE_SIZE)).astype(jnp.int32)
        g = (h << 2) | t
        outs[t][...] = g.reshape((_N,))


def _hash_ids_tc(ids_t):
    shp = jax.ShapeDtypeStruct((_N,), jnp.int32)
    return pl.pallas_call(
        _hash_body,
        out_shape=[shp, shp, shp, shp],
    )(ids_t)


def _sc_body(i0, i1, i2, i3, xv, out_hbm,
             slabs, idx_v, bufs, ssem, gsem, wsem):
    wid = lax.axis_index("s") * _NC + lax.axis_index("c")
    base = wid * _PER_W
    b0w = wid * _BPW
    idxin = (i0, i1, i2, i3)
    tables = (xv, xv, xv, xv)

    lane = lax.iota(jnp.int32, 16)
    for sub in range(_SUB):
        b0 = b0w + sub * _BSUB
        descs = [
            pltpu.async_copy(
                idxin[t].at[:, pl.ds(b0, _BSUB)], slabs.at[t], ssem
            )
            for t in range(4)
        ]
        for d in descs:
            d.wait()

        def trans_body(f, carry, sub=sub):
            for t in range(4):
                for bb in range(_BSUB // 16):
                    v = slabs[t, f, pl.ds(bb * 16, 16)]
                    nloc = (lane + sub * _BSUB + bb * 16) * _FIELDS + f
                    plsc.store_scatter(
                        idx_v,
                        [jnp.full((16,), t, jnp.int32),
                         nloc >> jnp.int32(7),
                         nloc & jnp.int32(127)],
                        v,
                    )
            return carry

        lax.fori_loop(0, _FIELDS, trans_body, 0)

    wait_dst = out_hbm.at[pl.ds(0, _G), pl.ds(0, _SUB_DIM)]

    def do_round(t, table, r, half, skip_wait):
        if not skip_wait:
            for b in range(4):
                pltpu.make_async_copy(bufs.at[half * 4 + b], wait_dst, wsem).wait()
        descs = []
        for b in range(4):
            g = r * 4 + b
            descs.append(
                pltpu.async_copy(table.at[idx_v.at[t, g]], bufs.at[half * 4 + b], gsem)
            )
        for d in descs:
            d.wait()
        for b in range(4):
            g = r * 4 + b
            row0 = base + g * _G
            pltpu.async_copy(
                bufs.at[half * 4 + b],
                out_hbm.at[pl.ds(row0, _G), pl.ds(t * _SUB_DIM, _SUB_DIM)],
                wsem,
            )

    nr = _NGRP // 4
    for t in range(4):
        table = tables[t]
        if t == 0:
            do_round(t, table, 0, 0, True)
            do_round(t, table, 1, 1, True)

            def body0(rr, carry):
                do_round(t, table, 2 + 2 * rr, 0, False)
                do_round(t, table, 3 + 2 * rr, 1, False)
                return carry

            lax.fori_loop(0, (nr - 2) // 2, body0, 0)
        else:

            def body(rr, carry, t=t, table=table):
                do_round(t, table, 2 * rr, 0, False)
                do_round(t, table, 2 * rr + 1, 1, False)
                return carry

            lax.fori_loop(0, nr // 2, body, 0)

    for b in range(8):
        pltpu.make_async_copy(bufs.at[b], wait_dst, wsem).wait()


def _sc_gather(idx2, xv):
    mesh = plsc.VectorSubcoreMesh(core_axis_name="c", subcore_axis_name="s")
    k = pl.kernel(
        _sc_body,
        out_type=jax.ShapeDtypeStruct((_N, _EMBED_DIM), jnp.float32),
        mesh=mesh,
        compiler_params=pltpu.CompilerParams(
            use_tc_tiling_on_sc=False, needs_layout_passes=False
        ),
        scratch_types=[
            pltpu.VMEM((4, _FIELDS, _BSUB), jnp.int32),
            pltpu.VMEM((4, _NGRP, _G), jnp.int32),
            pltpu.VMEM((8, _G, _SUB_DIM), jnp.float32),
            pltpu.SemaphoreType.DMA,
            pltpu.SemaphoreType.DMA,
            pltpu.SemaphoreType.DMA,
        ],
    )
    return k(*idx2, xv)


@jax.jit
def _bloom_embed(input_ids, t0, t1, t2, t3):
    idx1 = _hash_ids_tc(input_ids.T)
    idx2 = [v.reshape(_FIELDS, _BATCH) for v in idx1]
    xv = _repack4([t.T for t in (t0, t1, t2, t3)]).reshape(_XV_ROWS, _SUB_DIM)
    out = _sc_gather(idx2, xv)
    return out.reshape(_BATCH, _FIELDS, _EMBED_DIM)


def kernel(input_ids, table0, table1, table2, table3):
    return _bloom_embed(
        input_ids.astype(jnp.int32), table0, table1, table2, table3
    )


# speedup vs baseline: 1.2444x; 1.0134x over previous
import functools

import jax
import jax.numpy as jnp
from jax import lax
from jax.experimental import pallas as pl
from jax.experimental.pallas import tpu as pltpu
from jax.experimental.pallas import tpu_sc as plsc

_TABLE_SIZE = 1000000
_SUB_DIM = 32
_EMBED_DIM = 128
_HASH_SEED = 42

_BATCH = 16384
_FIELDS = 26
_N = _BATCH * _FIELDS

_NC = 2
_NS = 16
_NW = _NC * _NS
_PER_W = _N // _NW
_G = 128
_NGRP = _PER_W // _G
_BPW = _BATCH // _NW
_SUB = 2
_BSUB = _BPW // _SUB

_TR = 16384
_NBLK = 62
_XROWS = _NBLK * _TR
_XV_ROWS = 4 * _XROWS


def _repack_body(q0, q1, q2, q3, out_ref):
    out_ref[...] = jnp.concatenate([q0[...], q1[...], q2[...], q3[...]], axis=0).T


def _repack4(tts):
    spec = pl.BlockSpec((_SUB_DIM, _TR), lambda i: (0, i))
    return pl.pallas_call(
        _repack_body,
        grid=(_NBLK,),
        in_specs=[spec] * 4,
        out_specs=pl.BlockSpec((_TR, _EMBED_DIM), lambda i: (i, 0)),
        out_shape=jax.ShapeDtypeStruct((_XROWS, _EMBED_DIM), jnp.float32),
    )(*tts)


def _hash_body(ids_ref, o0, o1, o2, o3):
    x = ids_ref[...].astype(jnp.uint32)
    outs = (o0, o1, o2, o3)
    for t in range(4):
        h = x ^ jnp.uint32(_HASH_SEED + t)
        h = h * jnp.uint32(2654435761)
        h = h ^ (h >> jnp.uint32(16))
        h = h * jnp.uint32(2246822519)
        h = h ^ (h >> jnp.uint32(13))
        h = (h % jnp.uint32(_TABLE_SIZE)).astype(jnp.int32)
        g = (h << 2) | t
        outs[t][...] = g.reshape((_N,))


def _hash_ids_tc(ids_t):
    shp = jax.ShapeDtypeStruct((_N,), jnp.int32)
    return pl.pallas_call(
        _hash_body,
        out_shape=[shp, shp, shp, shp],
    )(ids_t)


def _sc_body(i0, i1, i2, i3, xv, out_hbm,
             slabs, idx_v, bufs, ssem, gsem, wsem):
    wid = lax.axis_index("s") * _NC + lax.axis_index("c")
    base = wid * _PER_W
    b0w = wid * _BPW
    idxin = (i0, i1, i2, i3)
    tables = (xv, xv, xv, xv)

    lane = lax.iota(jnp.int32, 16)
    for sub in range(_SUB):
        b0 = b0w + sub * _BSUB
        descs = [
            pltpu.async_copy(
                idxin[t].at[:, pl.ds(b0, _BSUB)], slabs.at[t], ssem
            )
            for t in range(4)
        ]
        for d in descs:
            d.wait()

        def trans_body(f, carry, sub=sub):
            for t in range(4):
                for bb in range(_BSUB // 16):
                    v = slabs[t, f, pl.ds(bb * 16, 16)]
                    nloc = (lane + sub * _BSUB + bb * 16) * _FIELDS + f
                    plsc.store_scatter(
                        idx_v,
                        [jnp.full((16,), t, jnp.int32),
                         nloc >> jnp.int32(7),
                         nloc & jnp.int32(127)],
                        v,
                    )
            return carry

        lax.fori_loop(0, _FIELDS, trans_body, 0)

    wait_dst = out_hbm.at[pl.ds(0, _G), pl.ds(0, _SUB_DIM)]

    def do_round(t, table, r, half, skip_wait):
        if not skip_wait:
            for b in range(4):
                pltpu.make_async_copy(bufs.at[half * 4 + b], wait_dst, wsem).wait()
        descs = []
        for b in range(4):
            g = r * 4 + b
            descs.append(
                pltpu.async_copy(table.at[idx_v.at[t, g]], bufs.at[half * 4 + b], gsem)
            )
        for d in descs:
            d.wait()
        for b in range(4):
            g = r * 4 + b
            row0 = base + g * _G
            pltpu.async_copy(
                bufs.at[half * 4 + b],
                out_hbm.at[pl.ds(row0, _G), pl.ds(t * _SUB_DIM, _SUB_DIM)],
                wsem,
            )

    nr = _NGRP // 4
    for t in range(4):
        table = tables[t]
        if t == 0:
            do_round(t, table, 0, 0, True)
            do_round(t, table, 1, 1, True)

            def body0(rr, carry):
                do_round(t, table, 2 + 2 * rr, 0, False)
                do_round(t, table, 3 + 2 * rr, 1, False)
                return carry

            lax.fori_loop(0, (nr - 2) // 2, body0, 0)
        else:

            def body(rr, carry, t=t, table=table):
                do_round(t, table, 2 * rr, 0, False)
                do_round(t, table, 2 * rr + 1, 1, False)
                return carry

            lax.fori_loop(0, nr // 2, body, 0)

    for b in range(8):
        pltpu.make_async_copy(bufs.at[b], wait_dst, wsem).wait()


def _sc_gather(idx2, xv):
    mesh = plsc.VectorSubcoreMesh(core_axis_name="c", subcore_axis_name="s")
    k = pl.kernel(
        _sc_body,
        out_type=jax.ShapeDtypeStruct((_N, _EMBED_DIM), jnp.float32),
        mesh=mesh,
        compiler_params=pltpu.CompilerParams(
            use_tc_tiling_on_sc=False, needs_layout_passes=False
        ),
        scratch_types=[
            pltpu.VMEM((4, _FIELDS, _BSUB), jnp.int32),
            pltpu.VMEM((4, _NGRP, _G), jnp.int32),
            pltpu.VMEM((8, _G, _SUB_DIM), jnp.float32),
            pltpu.SemaphoreType.DMA,
            pltpu.SemaphoreType.DMA,
            pltpu.SemaphoreType.DMA,
        ],
    )
    return k(*idx2, xv)


@jax.jit
def _bloom_embed(input_ids, t0, t1, t2, t3):
    idx1 = _hash_ids_tc(input_ids.T)
    idx2 = [v.reshape(_FIELDS, _BATCH) for v in idx1]
    xv = _repack4([t.T for t in (t0, t1, t2, t3)]).reshape(_XV_ROWS, _SUB_DIM)
    out = _sc_gather(idx2, xv)
    return out.reshape(_BATCH, _FIELDS, _EMBED_DIM)


def kernel(input_ids, table0, table1, table2, table3):
    return _bloom_embed(
        input_ids.astype(jnp.int32), table0, table1, table2, table3
    )


# speedup vs baseline: 1.2585x; 1.0114x over previous
import functools

import jax
import jax.numpy as jnp
from jax import lax
from jax.experimental import pallas as pl
from jax.experimental.pallas import tpu as pltpu
from jax.experimental.pallas import tpu_sc as plsc

_TABLE_SIZE = 1000000
_SUB_DIM = 32
_EMBED_DIM = 128
_HASH_SEED = 42

_BATCH = 16384
_FIELDS = 26
_N = _BATCH * _FIELDS

_NC = 2
_NS = 16
_NW = _NC * _NS
_PER_W = _N // _NW
_G = 128
_NGRP = _PER_W // _G
_BPW = _BATCH // _NW
_SUB = 2
_BSUB = _BPW // _SUB

_TR = 16384
_NBLK = 62
_XROWS = _NBLK * _TR
_XV_ROWS = 4 * _XROWS


def _repack_body(q0, q1, q2, q3, out_ref):
    out_ref[...] = jnp.concatenate([q0[...], q1[...], q2[...], q3[...]], axis=0).T


def _repack4(tts):
    spec = pl.BlockSpec((_SUB_DIM, _TR), lambda i: (0, i))
    return pl.pallas_call(
        _repack_body,
        grid=(_NBLK,),
        in_specs=[spec] * 4,
        out_specs=pl.BlockSpec((_TR, _EMBED_DIM), lambda i: (i, 0)),
        out_shape=jax.ShapeDtypeStruct((_XROWS, _EMBED_DIM), jnp.float32),
    )(*tts)


def _hash_body(ids_ref, o0, o1, o2, o3):
    x = ids_ref[...].astype(jnp.uint32)
    outs = (o0, o1, o2, o3)
    for t in range(4):
        h = x ^ jnp.uint32(_HASH_SEED + t)
        h = h * jnp.uint32(2654435761)
        h = h ^ (h >> jnp.uint32(16))
        h = h * jnp.uint32(2246822519)
        h = h ^ (h >> jnp.uint32(13))
        h = (h % jnp.uint32(_TABLE_SIZE)).astype(jnp.int32)
        g = (h << 2) | t
        outs[t][...] = g.reshape((_N,))


def _hash_ids_tc(ids_t):
    shp = jax.ShapeDtypeStruct((_N,), jnp.int32)
    return pl.pallas_call(
        _hash_body,
        out_shape=[shp, shp, shp, shp],
    )(ids_t)


def _sc_prep_body(i0, i1, i2, i3, l0, l1, l2, l3, slabs, idx_v, ssem):
    wid = lax.axis_index("s") * _NC + lax.axis_index("c")
    b0w = wid * _BPW
    idxin = (i0, i1, i2, i3)
    lists = (l0, l1, l2, l3)

    lane = lax.iota(jnp.int32, 16)
    for sub in range(_SUB):
        b0 = b0w + sub * _BSUB
        descs = [
            pltpu.async_copy(
                idxin[t].at[:, pl.ds(b0, _BSUB)], slabs.at[t], ssem
            )
            for t in range(4)
        ]
        for d in descs:
            d.wait()

        def trans_body(f, carry, sub=sub):
            for t in range(4):
                for bb in range(_BSUB // 16):
                    v = slabs[t, f, pl.ds(bb * 16, 16)]
                    nloc = (lane + sub * _BSUB + bb * 16) * _FIELDS + f
                    plsc.store_scatter(
                        idx_v,
                        [jnp.full((16,), t, jnp.int32),
                         nloc >> jnp.int32(7),
                         nloc & jnp.int32(127)],
                        v,
                    )
            return carry

        lax.fori_loop(0, _FIELDS, trans_body, 0)

    for t in range(4):
        pltpu.sync_copy(idx_v.at[t], lists[t].at[pl.ds(wid * _NGRP, _NGRP), :])


def _sc_prep(idx2):
    mesh = plsc.VectorSubcoreMesh(core_axis_name="c", subcore_axis_name="s")
    shp = jax.ShapeDtypeStruct((_NW * _NGRP, _G), jnp.int32)
    k = pl.kernel(
        _sc_prep_body,
        out_type=[shp] * 4,
        mesh=mesh,
        compiler_params=pltpu.CompilerParams(
            use_tc_tiling_on_sc=False, needs_layout_passes=False
        ),
        scratch_types=[
            pltpu.VMEM((4, _FIELDS, _BSUB), jnp.int32),
            pltpu.VMEM((4, _NGRP, _G), jnp.int32),
            pltpu.SemaphoreType.DMA,
        ],
    )
    return k(*idx2)


def _sc_body(l0, l1, l2, l3, xv, out_hbm, idx_v, bufs, ssem, gsem, wsem):
    wid = lax.axis_index("s") * _NC + lax.axis_index("c")
    base = wid * _PER_W
    tables = (xv, xv, xv, xv)

    for t, lst in enumerate((l0, l1, l2, l3)):
        pltpu.async_copy(
            lst.at[pl.ds(wid * _NGRP, _NGRP), :], idx_v.at[t], ssem
        )
    for t, lst in enumerate((l0, l1, l2, l3)):
        pltpu.make_async_copy(
            lst.at[pl.ds(wid * _NGRP, _NGRP), :], idx_v.at[t], ssem
        ).wait()

    wait_dst = out_hbm.at[pl.ds(0, _G), pl.ds(0, _SUB_DIM)]

    def do_round(t, table, r, half, skip_wait):
        if not skip_wait:
            for b in range(4):
                pltpu.make_async_copy(bufs.at[half * 4 + b], wait_dst, wsem).wait()
        descs = []
        for b in range(4):
            g = r * 4 + b
            descs.append(
                pltpu.async_copy(table.at[idx_v.at[t, g]], bufs.at[half * 4 + b], gsem)
            )
        for d in descs:
            d.wait()
        for b in range(4):
            g = r * 4 + b
            row0 = base + g * _G
            pltpu.async_copy(
                bufs.at[half * 4 + b],
                out_hbm.at[pl.ds(row0, _G), pl.ds(t * _SUB_DIM, _SUB_DIM)],
                wsem,
            )

    nr = _NGRP // 4
    for t in range(4):
        table = tables[t]
        if t == 0:
            do_round(t, table, 0, 0, True)
            do_round(t, table, 1, 1, True)

            def body0(rr, carry):
                do_round(t, table, 2 + 2 * rr, 0, False)
                do_round(t, table, 3 + 2 * rr, 1, False)
                return carry

            lax.fori_loop(0, (nr - 2) // 2, body0, 0)
        else:

            def body(rr, carry, t=t, table=table):
                do_round(t, table, 2 * rr, 0, False)
                do_round(t, table, 2 * rr + 1, 1, False)
                return carry

            lax.fori_loop(0, nr // 2, body, 0)

    for b in range(8):
        pltpu.make_async_copy(bufs.at[b], wait_dst, wsem).wait()


def _sc_gather(lists, xv):
    mesh = plsc.VectorSubcoreMesh(core_axis_name="c", subcore_axis_name="s")
    k = pl.kernel(
        _sc_body,
        out_type=jax.ShapeDtypeStruct((_N, _EMBED_DIM), jnp.float32),
        mesh=mesh,
        compiler_params=pltpu.CompilerParams(
            use_tc_tiling_on_sc=False, needs_layout_passes=False
        ),
        scratch_types=[
            pltpu.VMEM((4, _NGRP, _G), jnp.int32),
            pltpu.VMEM((8, _G, _SUB_DIM), jnp.float32),
            pltpu.SemaphoreType.DMA,
            pltpu.SemaphoreType.DMA,
            pltpu.SemaphoreType.DMA,
        ],
    )
    return k(*lists, xv)


@jax.jit
def _bloom_embed(input_ids, t0, t1, t2, t3):
    idx1 = _hash_ids_tc(input_ids.T)
    idx2 = [v.reshape(_FIELDS, _BATCH) for v in idx1]
    lists = _sc_prep(idx2)
    xv = _repack4([t.T for t in (t0, t1, t2, t3)]).reshape(_XV_ROWS, _SUB_DIM)
    out = _sc_gather(lists, xv)
    return out.reshape(_BATCH, _FIELDS, _EMBED_DIM)


def kernel(input_ids, table0, table1, table2, table3):
    return _bloom_embed(
        input_ids.astype(jnp.int32), table0, table1, table2, table3
    )
